# Initial kernel scaffold; baseline (speedup 1.0000x reference)
import jax
import jax.numpy as jnp
from jax.experimental import pallas as pl


def kernel(demand_seq_emb, supply_seq_emb, l, t_s, t_e, g_d_edge_index, g_d_edge_attr, comm, skill_semantic_embed, init_emb, skill_emb_1_weight, fuse_seq_W, fuse_seq_b, gnn0_Ws, gnn0_bs, gnn1_Ws, gnn1_bs):
    raise NotImplementedError("write your pallas kernel here")


# speedup vs baseline: 12.9807x; 12.9807x over previous
import functools

import jax
import jax.numpy as jnp
from jax import lax
from jax.experimental import pallas as pl
from jax.experimental.pallas import tpu as pltpu
from jax.experimental.pallas import tpu_sc as plsc

_N = 10000
_D = 128
_E = 320000
_P = 0.1

_BLK = 200
_NBLK = _N // _BLK

_NCORE = 2
_NSUB = 16
_NW = _NCORE * _NSUB
_LANE = 128
_CH = 79
_EW = _CH * _LANE
_EP = _NW * _EW
_RPT = _N // _NSUB


def _u_proj(uin, W, b):
    def body(uin_ref, w_ref, b_ref, o_ref):
        o_ref[...] = (
            jnp.dot(uin_ref[...], w_ref[...], preferred_element_type=jnp.float32)
            + b_ref[...]
        )
    return pl.pallas_call(
        body,
        out_shape=jax.ShapeDtypeStruct((_N, _D), jnp.float32),
    )(uin, W, b)


def _adj_pass(u):
    def body(ub_ref, ua_ref, pg_ref, dinv_ref, acc_ref):
        i = pl.program_id(0)
        logits = lax.dot_general(
            ub_ref[...], ua_ref[...], (((1,), (1,)), ((), ())),
            preferred_element_type=jnp.float32,
        )
        m = jnp.max(logits, axis=1, keepdims=True)
        e = jnp.exp(logits - m)
        p = e / jnp.sum(e, axis=1, keepdims=True)
        adj = jnp.maximum(p - 0.2, 0.0)
        pg_ref[...] = adj
        csum = jnp.sum(adj, axis=0, keepdims=True)

        @pl.when(i == 0)
        def _():
            acc_ref[...] = jnp.zeros_like(acc_ref)

        acc_ref[...] += csum

        @pl.when(i == _NBLK - 1)
        def _():
            dinv_ref[...] = lax.rsqrt(acc_ref[...] + 1.0)

    return pl.pallas_call(
        body,
        grid=(_NBLK,),
        in_specs=[
            pl.BlockSpec((_BLK, _D), lambda i: (i, 0)),
            pl.BlockSpec((_N, _D), lambda i: (0, 0)),
        ],
        out_specs=[
            pl.BlockSpec((_BLK, _N), lambda i: (i, 0)),
            pl.BlockSpec((1, _N), lambda i: (0, 0)),
        ],
        out_shape=[
            jax.ShapeDtypeStruct((_N, _N), jnp.float32),
            jax.ShapeDtypeStruct((1, _N), jnp.float32),
        ],
        scratch_shapes=[pltpu.VMEM((1, _N), jnp.float32)],
    )(u, u)


def _dense_gcn(pred_g, dinv_col, u, Ws, bs):
    def body(pg_ref, dinv_ref, u_ref, w_ref, b_ref, s0_ref,
             x_ref, xwd_ref, acc_ref):
        l = pl.program_id(0)
        j = pl.program_id(1)

        @pl.when(jnp.logical_and(l == 0, j == 0))
        def _():
            x_ref[...] = u_ref[...]

        @pl.when(j == 0)
        def _():
            xw = jnp.dot(x_ref[...], w_ref[...][0],
                         preferred_element_type=jnp.float32)
            xwd_ref[...] = xw * dinv_ref[...]
            acc_ref[...] = jnp.zeros_like(acc_ref)

        xwd_blk = xwd_ref[pl.ds(j * _BLK, _BLK), :]
        acc_ref[...] += lax.dot_general(
            pg_ref[...], xwd_blk, (((0,), (0,)), ((), ())),
            preferred_element_type=jnp.float32,
        )

        @pl.when(j == _NBLK - 1)
        def _():
            out = dinv_ref[...] * (acc_ref[...] + xwd_ref[...]) + b_ref[...][0]
            newx = (1.0 - _P) * out + _P * x_ref[...]
            x_ref[...] = newx

            @pl.when(l == 1)
            def _():
                s0_ref[...] = newx

    return pl.pallas_call(
        body,
        grid=(2, _NBLK),
        in_specs=[
            pl.BlockSpec((_BLK, _N), lambda l, j: (j, 0)),
            pl.BlockSpec((_N, 1), lambda l, j: (0, 0)),
            pl.BlockSpec((_N, _D), lambda l, j: (0, 0)),
            pl.BlockSpec((1, _D, _D), lambda l, j: (l, 0, 0)),
            pl.BlockSpec((1, 1, _D), lambda l, j: (l, 0, 0)),
        ],
        out_specs=pl.BlockSpec((_N, _D), lambda l, j: (0, 0)),
        out_shape=jax.ShapeDtypeStruct((_N, _D), jnp.float32),
        scratch_shapes=[
            pltpu.VMEM((_N, _D), jnp.float32),
            pltpu.VMEM((_N, _D), jnp.float32),
            pltpu.VMEM((_N, _D), jnp.float32),
        ],
    )(pred_g, dinv_col, u, Ws, bs)


def _z_first(u, W, degp_t):
    def body(u_ref, w_ref, dp_ref, z_ref, dinv_ref):
        dinv = lax.rsqrt(dp_ref[:, 0:1] + dp_ref[:, 1:2] + 1.0)
        dinv_ref[...] = dinv
        z_ref[...] = jnp.dot(u_ref[...], w_ref[...],
                             preferred_element_type=jnp.float32) * dinv
    return pl.pallas_call(
        body,
        out_shape=[
            jax.ShapeDtypeStruct((_N, _D), jnp.float32),
            jax.ShapeDtypeStruct((_N, 1), jnp.float32),
        ],
    )(u, W, degp_t)


def _z_mid(y00, y01, z0, xprev, dinv_s, W, b):
    def body(y0_ref, y1_ref, z_ref, x_ref, dinv_ref, w_ref, b_ref,
             zn_ref, t_ref):
        dinv = dinv_ref[...]
        out = dinv * (y0_ref[...] + y1_ref[...] + z_ref[...]) + b_ref[...]
        temp = (1.0 - _P) * out + _P * x_ref[...]
        t_ref[...] = temp
        zn_ref[...] = jnp.dot(temp, w_ref[...],
                              preferred_element_type=jnp.float32) * dinv
    return pl.pallas_call(
        body,
        out_shape=[
            jax.ShapeDtypeStruct((_N, _D), jnp.float32),
            jax.ShapeDtypeStruct((_N, _D), jnp.float32),
        ],
    )(y00, y01, z0, xprev, dinv_s, W, b)


def _final(s0, y10, y11, z1, temp1, dinv_s, b):
    def body(s0_ref, y0_ref, y1_ref, z_ref, t_ref, dinv_ref, b_ref, o_ref):
        out = dinv_ref[...] * (y0_ref[...] + y1_ref[...] + z_ref[...]) + b_ref[...]
        s1 = (1.0 - _P) * out + _P * t_ref[...]
        o_ref[...] = s0_ref[...] + s1
    return pl.pallas_call(
        body,
        out_shape=jax.ShapeDtypeStruct((_N, _D), jnp.float32),
    )(s0, y10, y11, z1, temp1, dinv_s, b)


_MESH = dict(core_axis_name="c", subcore_axis_name="s")


def _sc_deg(dst_p, w_p):
    @functools.partial(
        pl.kernel,
        mesh=plsc.VectorSubcoreMesh(**_MESH),
        out_type=jax.ShapeDtypeStruct((_NCORE, _N), jnp.float32),
        scratch_types=[
            pltpu.VMEM((_CH, _LANE), jnp.int32),
            pltpu.VMEM((_CH, _LANE), jnp.float32),
            pltpu.VMEM((_LANE,), jnp.float32),
            pltpu.VMEM_SHARED((_N,), jnp.float32),
            pltpu.SemaphoreType.DMA,
        ],
    )
    def k(dst_hbm, w_hbm, out_hbm, didx, wv, zv, deg_sh, sem):
        c = lax.axis_index("c")
        s = lax.axis_index("s")
        wid = c * _NSUB + s
        pltpu.sync_copy(dst_hbm.at[wid], didx)
        pltpu.sync_copy(w_hbm.at[wid], wv)
        for i in range(_LANE // 16):
            zv[pl.ds(i * 16, 16)] = jnp.zeros((16,), jnp.float32)

        @pl.when(s == 0)
        def _():
            def zloop(j, carry):
                pltpu.sync_copy(zv, deg_sh.at[pl.ds(j * _LANE, _LANE)])
                return carry
            lax.fori_loop(0, _N // _LANE, zloop, 0)
            pltpu.sync_copy(zv.at[pl.ds(0, 16)],
                            deg_sh.at[pl.ds((_N // _LANE) * _LANE, 16)])

        plsc.subcore_barrier()

        def body(j, carry):
            pltpu.sync_copy(wv.at[j], deg_sh.at[didx.at[j]], add=True)
            return carry
        lax.fori_loop(0, _CH, body, 0)

        plsc.subcore_barrier()

        @pl.when(s == 0)
        def _():
            pltpu.sync_copy(deg_sh, out_hbm.at[c])

    return k(dst_p, w_p)


def _sc_scatter(z, src_p, dst_p, w_p):
    @functools.partial(
        pl.kernel,
        mesh=plsc.VectorSubcoreMesh(**_MESH),
        out_type=jax.ShapeDtypeStruct((_NCORE, _N, _D), jnp.float32),
        scratch_types=[
            pltpu.VMEM((_CH, _LANE), jnp.int32),
            pltpu.VMEM((_CH, _LANE), jnp.int32),
            pltpu.VMEM((_CH, _LANE), jnp.float32),
            pltpu.VMEM((_LANE, _D), jnp.float32),
            pltpu.VMEM_SHARED((_N, _D), jnp.float32),
            pltpu.SemaphoreType.DMA,
        ],
    )
    def k(z_hbm, src_hbm, dst_hbm, w_hbm, out_hbm,
          sidx, didx, wv, rows, acc_sh, sem):
        c = lax.axis_index("c")
        s = lax.axis_index("s")
        wid = c * _NSUB + s
        pltpu.sync_copy(src_hbm.at[wid], sidx)
        pltpu.sync_copy(dst_hbm.at[wid], didx)
        pltpu.sync_copy(w_hbm.at[wid], wv)

        def zrow(r, carry):
            for cc in range(_D // 16):
                rows[r, pl.ds(cc * 16, 16)] = jnp.zeros((16,), jnp.float32)
            return carry
        lax.fori_loop(0, _LANE, zrow, 0)

        base = s * 624
        for kk in range(4):
            pltpu.sync_copy(rows, acc_sh.at[pl.ds(base + kk * _LANE, _LANE)])
        pltpu.sync_copy(rows.at[pl.ds(0, 112)],
                        acc_sh.at[pl.ds(base + 512, 112)])

        @pl.when(s == _NSUB - 1)
        def _():
            pltpu.sync_copy(rows.at[pl.ds(0, 16)],
                            acc_sh.at[pl.ds(9984, 16)])

        plsc.subcore_barrier()

        def body(j, carry):
            pltpu.async_copy(z_hbm.at[sidx.at[j]], rows, sem).wait()

            def scale(g, c2):
                wvec = wv[j, pl.ds(g * 16, 16)]
                for ee in range(16):
                    ws = wvec[ee]
                    e = g * 16 + ee
                    for cc in range(_D // 16):
                        sl = pl.ds(cc * 16, 16)
                        rows[e, sl] = rows[e, sl] * ws
                return c2
            lax.fori_loop(0, _LANE // 16, scale, 0)

            pltpu.sync_copy(rows, acc_sh.at[didx.at[j]], add=True)
            return carry
        lax.fori_loop(0, _CH, body, 0)

        plsc.subcore_barrier()
        pltpu.sync_copy(acc_sh.at[pl.ds(base, 624)],
                        out_hbm.at[c, pl.ds(base, 624)])

        @pl.when(s == _NSUB - 1)
        def _():
            pltpu.sync_copy(acc_sh.at[pl.ds(9984, 16)],
                            out_hbm.at[c, pl.ds(9984, 16)])

    return k(z, src_p, dst_p, w_p)


def kernel(demand_seq_emb, supply_seq_emb, l, t_s, t_e, g_d_edge_index,
           g_d_edge_attr, comm, skill_semantic_embed, init_emb,
           skill_emb_1_weight, fuse_seq_W, fuse_seq_b, gnn0_Ws, gnn0_bs,
           gnn1_Ws, gnn1_bs):
    uin = jnp.concatenate(
        [skill_emb_1_weight, demand_seq_emb[:, -1, :], supply_seq_emb[:, -1, :]],
        axis=1)
    u = _u_proj(uin, fuse_seq_W, fuse_seq_b.reshape(1, _D))

    pred_g, dinv_row = _adj_pass(u)
    dinv_col = dinv_row.reshape(_N, 1)
    s0 = _dense_gcn(pred_g, dinv_col, u, gnn0_Ws,
                    gnn0_bs.reshape(2, 1, _D))

    pad = _EP - _E
    fill = (jnp.arange(pad, dtype=jnp.int32) * 37) % _N
    src_p = jnp.concatenate([g_d_edge_index[0].astype(jnp.int32), fill]
                            ).reshape(_NW, _CH, _LANE)
    dst_p = jnp.concatenate([g_d_edge_index[1].astype(jnp.int32), fill]
                            ).reshape(_NW, _CH, _LANE)
    w_p = jnp.concatenate([g_d_edge_attr,
                           jnp.zeros((pad,), jnp.float32)]
                          ).reshape(_NW, _CH, _LANE)

    degp = _sc_deg(dst_p, w_p)
    z0, dinv_s = _z_first(u, gnn1_Ws[0], degp.T)
    y0 = _sc_scatter(z0, src_p, dst_p, w_p)
    z1, temp1 = _z_mid(y0[0], y0[1], z0, u, dinv_s,
                       gnn1_Ws[1], gnn1_bs[0].reshape(1, _D))
    y1 = _sc_scatter(z1, src_p, dst_p, w_p)
    skill_embs = _final(s0, y1[0], y1[1], z1, temp1, dinv_s,
                        gnn1_bs[1].reshape(1, _D))

    loss = jnp.zeros((), jnp.float32)
    return (skill_emb_1_weight, skill_embs, pred_g, loss)


# speedup vs baseline: 14.2656x; 1.0990x over previous
import functools

import jax
import jax.numpy as jnp
from jax import lax
from jax.experimental import pallas as pl
from jax.experimental.pallas import tpu as pltpu
from jax.experimental.pallas import tpu_sc as plsc

_N = 10000
_D = 128
_E = 320000
_P = 0.1

_BLK = 200
_NBLK = _N // _BLK

_NCORE = 2
_NSUB = 16
_NW = _NCORE * _NSUB
_LANE = 128
_CE = 96
_CH = 105
_SCH = 21
_EW = _CH * _CE
_EP = _NW * _EW
_RPT = _N // _NSUB


def _u_proj(uin, W, b):
    def body(uin_ref, w_ref, b_ref, o_ref):
        o_ref[...] = (
            jnp.dot(uin_ref[...], w_ref[...], preferred_element_type=jnp.float32)
            + b_ref[...]
        )
    return pl.pallas_call(
        body,
        out_shape=jax.ShapeDtypeStruct((_N, _D), jnp.float32),
    )(uin, W, b)


def _adj_pass(u):
    def body(ub_ref, ua_ref, pg_ref, dinv_ref, acc_ref):
        i = pl.program_id(0)
        logits = lax.dot_general(
            ub_ref[...], ua_ref[...], (((1,), (1,)), ((), ())),
            preferred_element_type=jnp.float32,
        )
        m = jnp.max(logits, axis=1, keepdims=True)
        e = jnp.exp(logits - m)
        p = e / jnp.sum(e, axis=1, keepdims=True)
        adj = jnp.maximum(p - 0.2, 0.0)
        pg_ref[...] = adj
        csum = jnp.sum(adj, axis=0, keepdims=True)

        @pl.when(i == 0)
        def _():
            acc_ref[...] = jnp.zeros_like(acc_ref)

        acc_ref[...] += csum

        @pl.when(i == _NBLK - 1)
        def _():
            dinv_ref[...] = lax.rsqrt(acc_ref[...] + 1.0)

    return pl.pallas_call(
        body,
        grid=(_NBLK,),
        in_specs=[
            pl.BlockSpec((_BLK, _D), lambda i: (i, 0)),
            pl.BlockSpec((_N, _D), lambda i: (0, 0)),
        ],
        out_specs=[
            pl.BlockSpec((_BLK, _N), lambda i: (i, 0)),
            pl.BlockSpec((1, _N), lambda i: (0, 0)),
        ],
        out_shape=[
            jax.ShapeDtypeStruct((_N, _N), jnp.float32),
            jax.ShapeDtypeStruct((1, _N), jnp.float32),
        ],
        scratch_shapes=[pltpu.VMEM((1, _N), jnp.float32)],
    )(u, u)


def _dense_gcn(pred_g, dinv_col, u, Ws, bs):
    def body(pg_ref, dinv_ref, u_ref, w_ref, b_ref, s0_ref,
             x_ref, xwd_ref, acc_ref):
        l = pl.program_id(0)
        j = pl.program_id(1)

        @pl.when(jnp.logical_and(l == 0, j == 0))
        def _():
            x_ref[...] = u_ref[...]

        @pl.when(j == 0)
        def _():
            xw = jnp.dot(x_ref[...], w_ref[...][0],
                         preferred_element_type=jnp.float32)
            xwd_ref[...] = xw * dinv_ref[...]
            acc_ref[...] = jnp.zeros_like(acc_ref)

        xwd_blk = xwd_ref[pl.ds(j * _BLK, _BLK), :]
        acc_ref[...] += lax.dot_general(
            pg_ref[...], xwd_blk, (((0,), (0,)), ((), ())),
            preferred_element_type=jnp.float32,
        )

        @pl.when(j == _NBLK - 1)
        def _():
            out = dinv_ref[...] * (acc_ref[...] + xwd_ref[...]) + b_ref[...][0]
            newx = (1.0 - _P) * out + _P * x_ref[...]
            x_ref[...] = newx

            @pl.when(l == 1)
            def _():
                s0_ref[...] = newx

    return pl.pallas_call(
        body,
        grid=(2, _NBLK),
        in_specs=[
            pl.BlockSpec((_BLK, _N), lambda l, j: (j, 0)),
            pl.BlockSpec((_N, 1), lambda l, j: (0, 0)),
            pl.BlockSpec((_N, _D), lambda l, j: (0, 0)),
            pl.BlockSpec((1, _D, _D), lambda l, j: (l, 0, 0)),
            pl.BlockSpec((1, 1, _D), lambda l, j: (l, 0, 0)),
        ],
        out_specs=pl.BlockSpec((_N, _D), lambda l, j: (0, 0)),
        out_shape=jax.ShapeDtypeStruct((_N, _D), jnp.float32),
        scratch_shapes=[
            pltpu.VMEM((_N, _D), jnp.float32),
            pltpu.VMEM((_N, _D), jnp.float32),
            pltpu.VMEM((_N, _D), jnp.float32),
        ],
    )(pred_g, dinv_col, u, Ws, bs)


def _z_first(u, W, degp_t):
    def body(u_ref, w_ref, dp_ref, z_ref, dinv_ref):
        dinv = lax.rsqrt(dp_ref[:, 0:1] + dp_ref[:, 1:2] + 1.0)
        dinv_ref[...] = dinv
        z_ref[...] = jnp.dot(u_ref[...], w_ref[...],
                             preferred_element_type=jnp.float32) * dinv
    return pl.pallas_call(
        body,
        out_shape=[
            jax.ShapeDtypeStruct((_N, _D), jnp.float32),
            jax.ShapeDtypeStruct((_N, 1), jnp.float32),
        ],
    )(u, W, degp_t)


def _z_mid(y00, y01, z0, xprev, dinv_s, W, b):
    def body(y0_ref, y1_ref, z_ref, x_ref, dinv_ref, w_ref, b_ref,
             zn_ref, t_ref):
        dinv = dinv_ref[...]
        out = dinv * (y0_ref[...] + y1_ref[...] + z_ref[...]) + b_ref[...]
        temp = (1.0 - _P) * out + _P * x_ref[...]
        t_ref[...] = temp
        zn_ref[...] = jnp.dot(temp, w_ref[...],
                              preferred_element_type=jnp.float32) * dinv
    return pl.pallas_call(
        body,
        out_shape=[
            jax.ShapeDtypeStruct((_N, _D), jnp.float32),
            jax.ShapeDtypeStruct((_N, _D), jnp.float32),
        ],
    )(y00, y01, z0, xprev, dinv_s, W, b)


def _final(s0, y10, y11, z1, temp1, dinv_s, b):
    def body(s0_ref, y0_ref, y1_ref, z_ref, t_ref, dinv_ref, b_ref, o_ref):
        out = dinv_ref[...] * (y0_ref[...] + y1_ref[...] + z_ref[...]) + b_ref[...]
        s1 = (1.0 - _P) * out + _P * t_ref[...]
        o_ref[...] = s0_ref[...] + s1
    return pl.pallas_call(
        body,
        out_shape=jax.ShapeDtypeStruct((_N, _D), jnp.float32),
    )(s0, y10, y11, z1, temp1, dinv_s, b)


_MESH = dict(core_axis_name="c", subcore_axis_name="s")


def _sc_deg(dst_p, w_p):
    @functools.partial(
        pl.kernel,
        mesh=plsc.VectorSubcoreMesh(**_MESH),
        out_type=jax.ShapeDtypeStruct((_NCORE, _N), jnp.float32),
        scratch_types=[
            pltpu.VMEM((_SCH, _CE), jnp.int32),
            pltpu.VMEM((_SCH, _CE), jnp.float32),
            pltpu.VMEM((_LANE,), jnp.float32),
            pltpu.VMEM_SHARED((_N,), jnp.float32),
            pltpu.SemaphoreType.DMA,
        ],
    )
    def k(dst_hbm, w_hbm, out_hbm, didx, wv, zv, deg_sh, sem):
        c = lax.axis_index("c")
        s = lax.axis_index("s")
        wid = c * _NSUB + s
        for i in range(_LANE // 16):
            zv[pl.ds(i * 16, 16)] = jnp.zeros((16,), jnp.float32)

        @pl.when(s == 0)
        def _():
            def zloop(j, carry):
                pltpu.sync_copy(zv, deg_sh.at[pl.ds(j * _LANE, _LANE)])
                return carry
            lax.fori_loop(0, _N // _LANE, zloop, 0)
            pltpu.sync_copy(zv.at[pl.ds(0, 16)],
                            deg_sh.at[pl.ds((_N // _LANE) * _LANE, 16)])

        plsc.subcore_barrier()

        for seg in range(_CH // _SCH):
            pltpu.sync_copy(dst_hbm.at[wid, seg], didx)
            pltpu.sync_copy(w_hbm.at[wid, seg], wv)

            def body(j, carry):
                pltpu.sync_copy(wv.at[j], deg_sh.at[didx.at[j]], add=True)
                return carry
            lax.fori_loop(0, _SCH, body, 0)

        plsc.subcore_barrier()

        @pl.when(s == 0)
        def _():
            pltpu.sync_copy(deg_sh, out_hbm.at[c])

    return k(dst_p, w_p)


def _sc_scatter(z, src_p, dst_p, w_p):
    @functools.partial(
        pl.kernel,
        mesh=plsc.VectorSubcoreMesh(**_MESH),
        out_type=jax.ShapeDtypeStruct((_NCORE, _N, _D), jnp.float32),
        scratch_types=[
            pltpu.VMEM((_SCH, _CE), jnp.int32),
            pltpu.VMEM((_SCH, _CE), jnp.int32),
            pltpu.VMEM((_SCH, _CE), jnp.float32),
            pltpu.VMEM((_CE, _D), jnp.float32),
            pltpu.VMEM((_CE, _D), jnp.float32),
            pltpu.VMEM((_CE, _D), jnp.float32),
            pltpu.VMEM_SHARED((_N, _D), jnp.float32),
            pltpu.SemaphoreType.DMA,
            pltpu.SemaphoreType.DMA,
            pltpu.SemaphoreType.DMA,
            pltpu.SemaphoreType.DMA,
            pltpu.SemaphoreType.DMA,
            pltpu.SemaphoreType.DMA,
        ],
    )
    def k(z_hbm, src_hbm, dst_hbm, w_hbm, out_hbm,
          sidx, didx, wv, rows0, rows1, rows2, acc_sh,
          g0, g1, g2, s0, s1, s2):
        c = lax.axis_index("c")
        s = lax.axis_index("s")
        wid = c * _NSUB + s
        rows_b = (rows0, rows1, rows2)
        gsem = (g0, g1, g2)
        ssem = (s0, s1, s2)

        def zrow(r, carry):
            for cc in range(_D // 16):
                rows0[r, pl.ds(cc * 16, 16)] = jnp.zeros((16,), jnp.float32)
            return carry
        lax.fori_loop(0, _CE, zrow, 0)

        base = s * 624
        for kk in range(6):
            pltpu.sync_copy(rows0, acc_sh.at[pl.ds(base + kk * _CE, _CE)])
        pltpu.sync_copy(rows0.at[pl.ds(0, 48)],
                        acc_sh.at[pl.ds(base + 576, 48)])

        @pl.when(s == _NSUB - 1)
        def _():
            pltpu.sync_copy(rows0.at[pl.ds(0, 16)],
                            acc_sh.at[pl.ds(9984, 16)])

        plsc.subcore_barrier()

        def _scale(jl, buf):
            def scale(g, c2):
                wvec = wv[jl, pl.ds(g * 16, 16)]
                for ee in range(16):
                    ws = wvec[ee]
                    e = g * 16 + ee
                    for cc in range(_D // 16):
                        sl = pl.ds(cc * 16, 16)
                        buf[e, sl] = buf[e, sl] * ws
                return c2
            lax.fori_loop(0, _CE // 16, scale, 0)

        for seg in range(_CH // _SCH):
            pltpu.sync_copy(src_hbm.at[wid, seg], sidx)
            pltpu.sync_copy(dst_hbm.at[wid, seg], didx)
            pltpu.sync_copy(w_hbm.at[wid, seg], wv)
            pltpu.async_copy(z_hbm.at[sidx.at[0]], rows0, g0)
            pltpu.async_copy(z_hbm.at[sidx.at[1]], rows1, g1)

            def body(jj, carry):
                for b in range(3):
                    jl = jj * 3 + b
                    buf = rows_b[b]
                    pltpu.make_async_copy(z_hbm.at[pl.ds(0, _CE)], buf,
                                          gsem[b]).wait()
                    _scale(jl, buf)
                    pltpu.async_copy(buf, acc_sh.at[didx.at[jl]], ssem[b],
                                     add=True)
                    bp = (b + 2) % 3

                    @pl.when(jl >= 1)
                    def _():
                        pltpu.make_async_copy(z_hbm.at[pl.ds(0, _CE)],
                                              rows_b[bp], ssem[bp]).wait()

                    @pl.when(jl < _SCH - 2)
                    def _():
                        pltpu.async_copy(z_hbm.at[sidx.at[jl + 2]],
                                         rows_b[bp], gsem[bp])
                return carry
            lax.fori_loop(0, _SCH // 3, body, 0)
            pltpu.make_async_copy(z_hbm.at[pl.ds(0, _CE)], rows2,
                                  ssem[2]).wait()

        plsc.subcore_barrier()
        pltpu.sync_copy(acc_sh.at[pl.ds(base, 624)],
                        out_hbm.at[c, pl.ds(base, 624)])

        @pl.when(s == _NSUB - 1)
        def _():
            pltpu.sync_copy(acc_sh.at[pl.ds(9984, 16)],
                            out_hbm.at[c, pl.ds(9984, 16)])

    return k(z, src_p, dst_p, w_p)


def kernel(demand_seq_emb, supply_seq_emb, l, t_s, t_e, g_d_edge_index,
           g_d_edge_attr, comm, skill_semantic_embed, init_emb,
           skill_emb_1_weight, fuse_seq_W, fuse_seq_b, gnn0_Ws, gnn0_bs,
           gnn1_Ws, gnn1_bs):
    uin = jnp.concatenate(
        [skill_emb_1_weight, demand_seq_emb[:, -1, :], supply_seq_emb[:, -1, :]],
        axis=1)
    u = _u_proj(uin, fuse_seq_W, fuse_seq_b.reshape(1, _D))

    pred_g, dinv_row = _adj_pass(u)
    dinv_col = dinv_row.reshape(_N, 1)
    s0 = _dense_gcn(pred_g, dinv_col, u, gnn0_Ws,
                    gnn0_bs.reshape(2, 1, _D))

    pad = _EP - _E
    nseg = _CH // _SCH
    fill = (jnp.arange(pad, dtype=jnp.int32) * 37) % _N
    src_p = jnp.concatenate([g_d_edge_index[0].astype(jnp.int32), fill]
                            ).reshape(_NW, nseg, _SCH, _CE)
    dst_p = jnp.concatenate([g_d_edge_index[1].astype(jnp.int32), fill]
                            ).reshape(_NW, nseg, _SCH, _CE)
    w_p = jnp.concatenate([g_d_edge_attr,
                           jnp.zeros((pad,), jnp.float32)]
                          ).reshape(_NW, nseg, _SCH, _CE)

    degp = _sc_deg(dst_p, w_p)
    z0, dinv_s = _z_first(u, gnn1_Ws[0], degp.T)
    y0 = _sc_scatter(z0, src_p, dst_p, w_p)
    z1, temp1 = _z_mid(y0[0], y0[1], z0, u, dinv_s,
                       gnn1_Ws[1], gnn1_bs[0].reshape(1, _D))
    y1 = _sc_scatter(z1, src_p, dst_p, w_p)
    skill_embs = _final(s0, y1[0], y1[1], z1, temp1, dinv_s,
                        gnn1_bs[1].reshape(1, _D))

    loss = jnp.zeros((), jnp.float32)
    return (skill_emb_1_weight, skill_embs, pred_g, loss)
